# Initial kernel scaffold; baseline (speedup 1.0000x reference)
import jax
import jax.numpy as jnp
from jax.experimental import pallas as pl


def kernel(feat, feat1, feat2, inds, inds1, inds2, wei1, wei2, dens_feat_f, dens_feat_s, W1, b1, W3, b3):
    raise NotImplementedError("write your pallas kernel here")


# speedup vs baseline: 2.5461x; 2.5461x over previous
import functools

import jax
import jax.numpy as jnp
from jax import lax
from jax.experimental import pallas as pl

NPTS = 500
PAD = 512
CF = 160
DIM = 64


def _fused_body(ft_ref, f1t_ref, f2t_ref, dfft_ref, dfst_ref,
                inds1_ref, inds_ref, inds2_ref, wei1_ref,
                w1t_ref, b1_ref, w3t_ref, b3_ref, out_ref):
    iota = lax.broadcasted_iota(jnp.int32, (PAD, PAD), 1)

    def accmat(idx, k, w=None, const=0.0):
        m = jnp.zeros((PAD, PAD), jnp.float32)
        for j in range(k):
            wj = w[:, j:j + 1] if w is not None else const
            m = m + jnp.where(idx[:, j:j + 1] == iota, wj, 0.0)
        return m

    wei1 = wei1_ref[...]
    inds1 = inds1_ref[...]
    at = accmat(inds1, 8, w=wei1 * 0.125)
    bt = accmat(inds_ref[...], 4, const=0.25)
    ct = accmat(inds2_ref[...], 8, const=0.125)
    dt = accmat(inds1[:, :4], 4, const=0.25)

    s1 = jnp.sum(wei1, axis=1, keepdims=True) * 0.125
    ft = ft_ref[...]
    fs1 = ft * s1
    pix = fs1 - jnp.dot(at, f1t_ref[...], preferred_element_type=jnp.float32)
    pt = fs1 - jnp.dot(at, f2t_ref[...], preferred_element_type=jnp.float32)
    plus = pix + jnp.dot(bt, pt, preferred_element_type=jnp.float32)
    ds = jax.nn.sigmoid(
        jnp.dot(plus, w1t_ref[...], preferred_element_type=jnp.float32)
        + b1_ref[...])
    m = jnp.dot(ct, ds, preferred_element_type=jnp.float32)
    new_f = dfft_ref[...] * m
    plus2 = dfst_ref[...] + jnp.dot(dt, new_f, preferred_element_type=jnp.float32)
    out_ref[...] = (jnp.dot(plus2, w3t_ref[...], preferred_element_type=jnp.float32)
                    + b3_ref[...])


@functools.partial(jax.jit, static_argnames=())
def kernel(feat, feat1, feat2, inds, inds1, inds2, wei1, wei2,
           dens_feat_f, dens_feat_s, W1, b1, W3, b3):
    del wei2
    padp = PAD - NPTS

    def padt(x):
        return jnp.pad(x[0].T, ((0, padp), (0, 0)))

    ft = padt(feat)
    f1t = padt(feat1)
    f2t = padt(feat2)
    dfft = padt(dens_feat_f)
    dfst = padt(dens_feat_s)

    def padi(x, k):
        return jnp.pad(x[0].astype(jnp.int32).reshape(NPTS, k), ((0, padp), (0, 0)))

    inds1r = padi(inds1, 8)
    indsr = padi(inds, 4)
    inds2r = padi(inds2, 8)
    wei1r = jnp.pad(wei1[0].reshape(NPTS, 8), ((0, padp), (0, 0)))

    out_t = pl.pallas_call(
        _fused_body,
        out_shape=jax.ShapeDtypeStruct((PAD, CF), jnp.float32),
    )(ft, f1t, f2t, dfft, dfst, inds1r, indsr, inds2r, wei1r,
      W1.T, b1[None, :], W3.T, b3[None, :])
    return out_t[:NPTS].T[None]
